# baseline (speedup 1.0000x reference)
import jax
import jax.numpy as jnp
from jax.experimental import pallas as pl

N_LT = 100000
N_BV = 100000
N_C = 100000
N_TOPIC = 10000
N_BILL = 50000
D = 128
E_VOTE = 100000


def _seg_mean(src, index, num_segments):
    sums = jax.ops.segment_sum(src, index, num_segments=num_segments)
    cnt = jax.ops.segment_sum(jnp.ones((src.shape[0],), src.dtype), index,
                              num_segments=num_segments)
    return sums / jnp.clip(cnt, 1.0)[:, None]


def _transform_kernel(hl_ref, hb_ref, hc_ref, w_ref, out_ref):
    w = w_ref[...]
    dn = (((1,), (1,)), ((), ()))
    out_ref[0] = jax.lax.dot_general(hl_ref[...], w[:, 0:128], dn)
    out_ref[1] = jax.lax.dot_general(hb_ref[...], w[:, 128:256], dn)
    out_ref[2] = jax.lax.dot_general(hc_ref[...], w[:, 256:384], dn)
    out_ref[3] = jax.lax.dot_general(hc_ref[...], w[:, 384:512], dn)


def _topic_kernel(ht_ref, w_ref, b_ref, out_ref):
    dn = (((1,), (1,)), ((), ()))
    out_ref[...] = jax.lax.dot_general(ht_ref[...], w_ref[...][:, 512:640], dn) + b_ref[...]


def _transforms(h_lt, h_bv, h_c, h_topic, W, b):
    RB = 1000
    t4 = pl.pallas_call(
        _transform_kernel,
        grid=(N_LT // RB,),
        in_specs=[
            pl.BlockSpec((RB, D), lambda i: (i, 0)),
            pl.BlockSpec((RB, D), lambda i: (i, 0)),
            pl.BlockSpec((RB, D), lambda i: (i, 0)),
            pl.BlockSpec((D, 5 * D), lambda i: (0, 0)),
        ],
        out_specs=pl.BlockSpec((4, RB, D), lambda i: (0, i, 0)),
        out_shape=jax.ShapeDtypeStruct((4, N_LT, D), jnp.float32),
    )(h_lt, h_bv, h_c, W)
    t5 = pl.pallas_call(
        _topic_kernel,
        grid=(N_TOPIC // RB,),
        in_specs=[
            pl.BlockSpec((RB, D), lambda i: (i, 0)),
            pl.BlockSpec((D, 5 * D), lambda i: (0, 0)),
            pl.BlockSpec((D,), lambda i: (0,)),
        ],
        out_specs=pl.BlockSpec((RB, D), lambda i: (i, 0)),
        out_shape=jax.ShapeDtypeStruct((N_TOPIC, D), jnp.float32),
    )(h_topic, W, b)
    return t4, t5


def kernel(h_legislator_term, h_bill_version, h_committee, h_topic, vote_edges,
           bv2b, topic_for_bill, prior_edge_src, read_edge_dst, member_edge_dst,
           W, b):
    lt_idx = vote_edges[0]
    bv_idx = vote_edges[1]
    bill_idx = jnp.take(bv2b, bv_idx, axis=0)
    topic_idx = jnp.take(topic_for_bill, bill_idx, axis=0)

    t4, t5 = _transforms(h_legislator_term, h_bill_version, h_committee,
                         h_topic, W, b)

    m1 = _seg_mean(t4[0], lt_idx, N_LT)
    m2 = _seg_mean(t4[1], prior_edge_src, N_BV)
    m3 = _seg_mean(t4[2], read_edge_dst, N_C)
    m4 = _seg_mean(t4[3], member_edge_dst, N_C)

    out = (jnp.take(m1, lt_idx, axis=0)
           + jnp.take(m2, bv_idx, axis=0)
           + jnp.take(m3, bill_idx, axis=0)
           + jnp.take(m4, lt_idx, axis=0)
           + jnp.take(t5, jnp.maximum(topic_idx, 0), axis=0))
    return out
